# baseline (speedup 1.0000x reference)
import jax
import jax.numpy as jnp
from jax import lax
from jax.experimental import pallas as pl
from jax.experimental.pallas import tpu as pltpu

NUM_VERBS = 25
NUM_HOIS = 234
HOIS_PAD = 256
BATCH = 16384
BLOCK_B = 8192
NBLK = BATCH // BLOCK_B


def _gather_kernel(idx_ref, in_ref, out_ref):
    verb_iota = lax.broadcasted_iota(jnp.int32, (NUM_VERBS, HOIS_PAD), 0)
    onehot = (idx_ref[0][None, :] == verb_iota).astype(jnp.float32)
    out_ref[...] = jnp.dot(
        in_ref[...], onehot, preferred_element_type=jnp.float32
    )


@jax.jit
def kernel(verb_scores, hoi_to_verb):
    idx_pad = jnp.full((HOIS_PAD,), -1, jnp.int32).at[:NUM_HOIS].set(hoi_to_verb)
    out = pl.pallas_call(
        _gather_kernel,
        grid=(NBLK,),
        in_specs=[
            pl.BlockSpec((1, HOIS_PAD), lambda i: (0, 0)),
            pl.BlockSpec((BLOCK_B, NUM_VERBS), lambda i: (i, 0)),
        ],
        out_specs=pl.BlockSpec((BLOCK_B, HOIS_PAD), lambda i: (i, 0)),
        out_shape=jax.ShapeDtypeStruct((BATCH, HOIS_PAD), jnp.float32),
        compiler_params=pltpu.CompilerParams(
            dimension_semantics=("parallel",),
        ),
    )(idx_pad.reshape(1, HOIS_PAD), verb_scores)
    return out[:, :NUM_HOIS]


# speedup vs baseline: 1.3190x; 1.3190x over previous
import jax
import jax.numpy as jnp
from jax import lax
from jax.experimental import pallas as pl
from jax.experimental.pallas import tpu as pltpu

NUM_VERBS = 25
NUM_HOIS = 234
BATCH = 16384
BLOCK_B = 8192
BLOCK_H = 128
NBLK_B = BATCH // BLOCK_B
NBLK_H = -(-NUM_HOIS // BLOCK_H)


def _gather_kernel(idx_ref, in_ref, out_ref):
    verb_iota = lax.broadcasted_iota(jnp.int32, (NUM_VERBS, BLOCK_H), 0)
    onehot = (idx_ref[0][None, :] == verb_iota).astype(jnp.float32)
    out_ref[...] = jnp.dot(
        in_ref[...], onehot, preferred_element_type=jnp.float32
    )


@jax.jit
def kernel(verb_scores, hoi_to_verb):
    return pl.pallas_call(
        _gather_kernel,
        grid=(NBLK_H, NBLK_B),
        in_specs=[
            pl.BlockSpec((1, BLOCK_H), lambda j, i: (0, j)),
            pl.BlockSpec((BLOCK_B, NUM_VERBS), lambda j, i: (i, 0)),
        ],
        out_specs=pl.BlockSpec((BLOCK_B, BLOCK_H), lambda j, i: (i, j)),
        out_shape=jax.ShapeDtypeStruct((BATCH, NUM_HOIS), jnp.float32),
        compiler_params=pltpu.CompilerParams(
            dimension_semantics=("arbitrary", "arbitrary"),
        ),
    )(hoi_to_verb.reshape(1, NUM_HOIS), verb_scores)


# speedup vs baseline: 1.3731x; 1.0410x over previous
import jax
import jax.numpy as jnp
from jax import lax
from jax.experimental import pallas as pl
from jax.experimental.pallas import tpu as pltpu

NUM_VERBS = 25
NUM_HOIS = 234
BATCH = 16384
BLOCK_B = 8192
BLOCK_H = 128
NBLK_B = BATCH // BLOCK_B
NBLK_H = -(-NUM_HOIS // BLOCK_H)


def _gather_kernel(idx_ref, in_ref, out_ref):
    verb_iota = lax.broadcasted_iota(jnp.int32, (NUM_VERBS, BLOCK_H), 0)
    onehot = (idx_ref[0][None, :] == verb_iota).astype(jnp.float32)
    out_ref[...] = jnp.dot(
        in_ref[...], onehot, preferred_element_type=jnp.float32
    )


@jax.jit
def kernel(verb_scores, hoi_to_verb):
    return pl.pallas_call(
        _gather_kernel,
        grid=(NBLK_B, NBLK_H),
        in_specs=[
            pl.BlockSpec((1, BLOCK_H), lambda i, j: (0, j)),
            pl.BlockSpec((BLOCK_B, NUM_VERBS), lambda i, j: (i, 0)),
        ],
        out_specs=pl.BlockSpec((BLOCK_B, BLOCK_H), lambda i, j: (i, j)),
        out_shape=jax.ShapeDtypeStruct((BATCH, NUM_HOIS), jnp.float32),
        compiler_params=pltpu.CompilerParams(
            dimension_semantics=("arbitrary", "arbitrary"),
        ),
    )(hoi_to_verb.reshape(1, NUM_HOIS), verb_scores)


# speedup vs baseline: 1.4074x; 1.0250x over previous
import jax
import jax.numpy as jnp
from jax import lax
from jax.experimental import pallas as pl
from jax.experimental.pallas import tpu as pltpu

NUM_VERBS = 25
NUM_HOIS = 234
BATCH = 16384
BLOCK_B = 8192
NBLK = BATCH // BLOCK_B


def _gather_kernel(idx_ref, in_ref, out_ref):
    i = pl.program_id(0)
    verb_iota = lax.broadcasted_iota(jnp.int32, (NUM_VERBS, NUM_HOIS), 0)
    onehot = (idx_ref[0][None, :] == verb_iota).astype(jnp.float32)
    block = in_ref[pl.ds(i * BLOCK_B, BLOCK_B), :]
    out_ref[...] = jnp.dot(block, onehot, preferred_element_type=jnp.float32)


@jax.jit
def kernel(verb_scores, hoi_to_verb):
    return pl.pallas_call(
        _gather_kernel,
        grid=(NBLK,),
        in_specs=[
            pl.BlockSpec((1, NUM_HOIS), lambda i: (0, 0)),
            pl.BlockSpec(memory_space=pltpu.MemorySpace.VMEM),
        ],
        out_specs=pl.BlockSpec((BLOCK_B, NUM_HOIS), lambda i: (i, 0)),
        out_shape=jax.ShapeDtypeStruct((BATCH, NUM_HOIS), jnp.float32),
        compiler_params=pltpu.CompilerParams(
            dimension_semantics=("arbitrary",),
        ),
    )(hoi_to_verb.reshape(1, NUM_HOIS), verb_scores)


# speedup vs baseline: 1.4664x; 1.0419x over previous
import jax
import jax.numpy as jnp
from jax import lax
from jax.experimental import pallas as pl
from jax.experimental.pallas import tpu as pltpu

NUM_VERBS = 25
NUM_HOIS = 234
BATCH = 16384
BLOCK_B = 8192
NBLK = BATCH // BLOCK_B


def _gather_via_onehot(idx_ref, in_ref, out_ref):
    verb_iota = lax.broadcasted_iota(jnp.int32, (NUM_VERBS, NUM_HOIS), 0)
    onehot = (idx_ref[0][None, :] == verb_iota).astype(jnp.float32)
    out_ref[...] = jnp.dot(
        in_ref[...], onehot, preferred_element_type=jnp.float32
    )


@jax.jit
def kernel(verb_scores, hoi_to_verb):
    return pl.pallas_call(
        _gather_via_onehot,
        grid=(NBLK,),
        in_specs=[
            pl.BlockSpec((1, NUM_HOIS), lambda i: (0, 0)),
            pl.BlockSpec((BLOCK_B, NUM_VERBS), lambda i: (i, 0)),
        ],
        out_specs=pl.BlockSpec((BLOCK_B, NUM_HOIS), lambda i: (i, 0)),
        out_shape=jax.ShapeDtypeStruct((BATCH, NUM_HOIS), jnp.float32),
        compiler_params=pltpu.CompilerParams(
            dimension_semantics=("parallel",),
        ),
    )(hoi_to_verb.reshape(1, NUM_HOIS), verb_scores)
